# Initial kernel scaffold; baseline (speedup 1.0000x reference)
import jax
import jax.numpy as jnp
from jax.experimental import pallas as pl


def kernel(inputs, target):
    raise NotImplementedError("write your pallas kernel here")


# speedup vs baseline: 138.9903x; 138.9903x over previous
import jax
import jax.numpy as jnp
from jax.experimental import pallas as pl
from jax.experimental.pallas import tpu as pltpu

_NC = 19
_RATIO = 0.2
_IGNORE = -1.0


def _ce_body(x_ref, t_ref, loss_ref, acc_ref, *, block_r, width):
    step = pl.program_id(0)
    nsteps = pl.num_programs(0)

    @pl.when(step == 0)
    def _init():
        acc_ref[...] = jnp.zeros_like(acc_ref)

    x = x_ref[...]
    t = t_ref[...]

    m = jnp.max(x, axis=0)
    cls = jax.lax.broadcasted_iota(jnp.int32, (_NC, block_r, width), 0)
    idx = jnp.min(jnp.where(x == m[None, :, :], cls, _NC), axis=0)

    lse = m + jnp.log(jnp.sum(jnp.exp(x - m[None, :, :]), axis=0))
    tm = jnp.where(t != _IGNORE, t, 0.0)
    a = jnp.sum(tm, axis=0)
    b = jnp.sum(tm * x, axis=0)
    g = lse * a - b

    onehot = idx[None, :, :] == cls
    cnt = jnp.sum(onehot.astype(jnp.float32), axis=(1, 2)).reshape(_NC, 1)
    gsum = jnp.sum(jnp.where(onehot, g[None, :, :], 0.0), axis=(1, 2)).reshape(_NC, 1)
    acc_ref[:, 0:1] += cnt
    acc_ref[:, 1:2] += gsum

    @pl.when(step == nsteps - 1)
    def _finish():
        hist = acc_ref[:, 0:1]
        gs = acc_ref[:, 1:2]
        total = jnp.sum(hist)
        denom = jnp.maximum(
            jnp.exp(_RATIO * jnp.log(hist) + (1.0 - _RATIO) * jnp.log(total)), 1.0
        )
        loss_ref[...] = (jnp.sum(gs / denom) / _NC).reshape(1, 1)


def kernel(inputs, target):
    n, nc, h, w = inputs.shape
    x = inputs.reshape(nc, h, w)
    t = target.reshape(nc, h, w)
    block_r = 32
    grid = h // block_r

    import functools

    body = functools.partial(_ce_body, block_r=block_r, width=w)
    out = pl.pallas_call(
        body,
        grid=(grid,),
        in_specs=[
            pl.BlockSpec((nc, block_r, w), lambda i: (0, i, 0)),
            pl.BlockSpec((nc, block_r, w), lambda i: (0, i, 0)),
        ],
        out_specs=pl.BlockSpec((1, 1), lambda i: (0, 0)),
        out_shape=jax.ShapeDtypeStruct((1, 1), jnp.float32),
        scratch_shapes=[pltpu.VMEM((_NC, 2), jnp.float32)],
    )(x, t)
    return out[0, 0]
